# Initial kernel scaffold; baseline (speedup 1.0000x reference)
import jax
import jax.numpy as jnp
from jax.experimental import pallas as pl


def kernel(input, embeddings):
    raise NotImplementedError("write your pallas kernel here")


# speedup vs baseline: 1.1307x; 1.1307x over previous
import functools

import jax
import jax.numpy as jnp
from jax import lax
from jax.experimental import pallas as pl
from jax.experimental.pallas import tpu as pltpu
from jax.experimental.pallas import tpu_sc as plsc

BETA = 0.25
N_TOK = 16384
D = 64
K = 8192
BR = 256


W = 4096


def _round_bf16(x):
    u = lax.bitcast_convert_type(x, jnp.uint32)
    r = u + jnp.uint32(0x7FFF) + ((u >> 16) & jnp.uint32(1))
    return lax.bitcast_convert_type(r & jnp.uint32(0xFFFF0000), jnp.float32)


def _argmin_body(f_ref, e_ref, a_ref, c_ref, idx_ref):
    f = f_ref[...]
    e = e_ref[...]
    mm = lax.dot_general(f, e, (((1,), (0,)), ((), ())),
                         preferred_element_type=jnp.float32)
    val = -((a_ref[...] - mm) + c_ref[...])
    accv = jnp.full((val.shape[0], 1), -jnp.inf, jnp.float32)
    acci = jnp.zeros((val.shape[0], 1), jnp.int32)
    for w in range(K // W):
        sub = val[:, w * W:(w + 1) * W]
        m = jnp.max(sub, axis=1, keepdims=True)
        ii = lax.broadcasted_iota(jnp.int32, sub.shape, 1)
        wi = jnp.min(jnp.where(sub == m, ii, K), axis=1, keepdims=True) + w * W
        bv = _round_bf16(accv)
        take = m > bv
        accv = jnp.where(take, m, bv)
        acci = jnp.where(take, wi, acci)
    idx_ref[...] = acci[:, 0]


_argmin = pl.pallas_call(
    _argmin_body,
    grid=(N_TOK // BR,),
    in_specs=[
        pl.BlockSpec((BR, D), lambda i: (i, 0)),
        pl.BlockSpec((D, K), lambda i: (0, 0)),
        pl.BlockSpec((BR, 1), lambda i: (i, 0)),
        pl.BlockSpec((1, K), lambda i: (0, 0)),
    ],
    out_specs=pl.BlockSpec((BR,), lambda i: (i,)),
    out_shape=jax.ShapeDtypeStruct((N_TOK,), jnp.int32),
)


def _loss_body(c_ref, f_ref, o_ref):
    d = c_ref[...] - f_ref[...]
    m = jnp.sum(d * d) / jnp.float32(N_TOK * D)
    o_ref[0, 0] = m + jnp.float32(BETA) * m


_loss = pl.pallas_call(
    _loss_body,
    in_specs=[
        pl.BlockSpec(memory_space=pltpu.VMEM),
        pl.BlockSpec(memory_space=pltpu.VMEM),
    ],
    out_specs=pl.BlockSpec(memory_space=pltpu.SMEM),
    out_shape=jax.ShapeDtypeStruct((1, 1), jnp.float32),
)


DPAD = 128


def _sc_gather(table, ind):
    mesh = plsc.VectorSubcoreMesh(core_axis_name="c", subcore_axis_name="s")
    nc, ns = mesh.num_cores, mesh.num_subcores
    nw = nc * ns
    bpw = N_TOK // nw

    @functools.partial(
        pl.kernel,
        mesh=mesh,
        out_type=jax.ShapeDtypeStruct((N_TOK, DPAD), jnp.float32),
        scratch_types=[
            pltpu.VMEM((bpw,), jnp.int32),
            pltpu.VMEM((bpw, DPAD), jnp.float32),
            pltpu.SemaphoreType.DMA,
        ],
    )
    def gather(table_hbm, idx_hbm, out_hbm, idx_v, rows_v, sem):
        wid = lax.axis_index("s") * nc + lax.axis_index("c")
        base = wid * bpw
        pltpu.sync_copy(idx_hbm.at[pl.ds(base, bpw)], idx_v)
        pltpu.async_copy(table_hbm.at[idx_v], rows_v, sem).wait()
        pltpu.sync_copy(rows_v, out_hbm.at[pl.ds(base, bpw)])

    return gather(table, ind)


def kernel(input, embeddings):
    x = jnp.transpose(input, (0, 2, 3, 1))
    flatten = x.reshape(-1, D)
    a = jnp.sum(flatten ** 2, axis=1, keepdims=True)
    c = jnp.sum(embeddings ** 2, axis=0, keepdims=True)
    f2b = (2.0 * flatten).astype(jnp.bfloat16).astype(jnp.float32)
    ind = _argmin(f2b, embeddings, a, c)
    table = jnp.pad(embeddings.T, ((0, 0), (0, DPAD - D)))
    codes = _sc_gather(table, ind)[:, :D]
    loss = _loss(codes, flatten)[0, 0]
    embed_ind = ind.reshape(x.shape[:-1])
    q = jnp.transpose(codes.reshape(x.shape), (0, 3, 1, 2))
    quantize_st = input + lax.stop_gradient(q - input)
    return (quantize_st, loss, embed_ind)


# speedup vs baseline: 1.2441x; 1.1003x over previous
import functools

import jax
import jax.numpy as jnp
from jax import lax
from jax.experimental import pallas as pl
from jax.experimental.pallas import tpu as pltpu
from jax.experimental.pallas import tpu_sc as plsc

BETA = 0.25
N_TOK = 16384
D = 64
K = 8192
BR = 512


W = 4096


def _round_bf16(x):
    u = lax.bitcast_convert_type(x, jnp.uint32)
    r = u + jnp.uint32(0x7FFF) + ((u >> 16) & jnp.uint32(1))
    return lax.bitcast_convert_type(r & jnp.uint32(0xFFFF0000), jnp.float32)


def _argmin_body(f_ref, e_ref, a_ref, c_ref, idx_ref):
    f = f_ref[...]
    e = e_ref[...]
    mm = lax.dot_general(f, e, (((1,), (0,)), ((), ())),
                         preferred_element_type=jnp.float32)
    dist = (a_ref[...] - mm) + c_ref[...]
    accv = jnp.full((dist.shape[0], 1), jnp.inf, jnp.float32)
    acci = jnp.zeros((dist.shape[0], 1), jnp.int32)
    for w in range(K // W):
        sub = dist[:, w * W:(w + 1) * W]
        m = jnp.min(sub, axis=1, keepdims=True)
        ii = lax.broadcasted_iota(jnp.int32, sub.shape, 1)
        wi = jnp.min(jnp.where(sub == m, ii, K), axis=1, keepdims=True) + w * W
        bv = _round_bf16(accv)
        take = m < bv
        accv = jnp.where(take, m, bv)
        acci = jnp.where(take, wi, acci)
    idx_ref[...] = acci[:, 0]


_argmin = pl.pallas_call(
    _argmin_body,
    grid=(N_TOK // BR,),
    in_specs=[
        pl.BlockSpec((BR, D), lambda i: (i, 0)),
        pl.BlockSpec((D, K), lambda i: (0, 0)),
        pl.BlockSpec((BR, 1), lambda i: (i, 0)),
        pl.BlockSpec((1, K), lambda i: (0, 0)),
    ],
    out_specs=pl.BlockSpec((BR,), lambda i: (i,)),
    out_shape=jax.ShapeDtypeStruct((N_TOK,), jnp.int32),
)


def _loss_body(c_ref, f_ref, o_ref):
    d = c_ref[...] - f_ref[...]
    m = jnp.sum(d * d) / jnp.float32(N_TOK * D)
    o_ref[0, 0] = m + jnp.float32(BETA) * m


_loss = pl.pallas_call(
    _loss_body,
    in_specs=[
        pl.BlockSpec(memory_space=pltpu.VMEM),
        pl.BlockSpec(memory_space=pltpu.VMEM),
    ],
    out_specs=pl.BlockSpec(memory_space=pltpu.SMEM),
    out_shape=jax.ShapeDtypeStruct((1, 1), jnp.float32),
)


DPAD = 128


def _sc_gather(table, ind):
    mesh = plsc.VectorSubcoreMesh(core_axis_name="c", subcore_axis_name="s")
    nc, ns = mesh.num_cores, mesh.num_subcores
    nw = nc * ns
    bpw = N_TOK // nw

    @functools.partial(
        pl.kernel,
        mesh=mesh,
        out_type=jax.ShapeDtypeStruct((N_TOK, DPAD), jnp.float32),
        scratch_types=[
            pltpu.VMEM((bpw,), jnp.int32),
            pltpu.VMEM((bpw, DPAD), jnp.float32),
            pltpu.SemaphoreType.DMA,
        ],
    )
    def gather(table_hbm, idx_hbm, out_hbm, idx_v, rows_v, sem):
        wid = lax.axis_index("s") * nc + lax.axis_index("c")
        base = wid * bpw
        pltpu.sync_copy(idx_hbm.at[pl.ds(base, bpw)], idx_v)
        pltpu.async_copy(table_hbm.at[idx_v], rows_v, sem).wait()
        pltpu.sync_copy(rows_v, out_hbm.at[pl.ds(base, bpw)])

    return gather(table, ind)


def kernel(input, embeddings):
    x = jnp.transpose(input, (0, 2, 3, 1))
    flatten = x.reshape(-1, D)
    a = jnp.sum(flatten ** 2, axis=1, keepdims=True)
    c = jnp.sum(embeddings ** 2, axis=0, keepdims=True)
    f2b = (2.0 * flatten).astype(jnp.bfloat16).astype(jnp.float32)
    ind = _argmin(f2b, embeddings, a, c)
    table = jnp.pad(embeddings.T, ((0, 0), (0, DPAD - D)))
    codes = _sc_gather(table, ind)[:, :D]
    loss = _loss(codes, flatten)[0, 0]
    embed_ind = ind.reshape(x.shape[:-1])
    q = jnp.transpose(codes.reshape(x.shape), (0, 3, 1, 2))
    quantize_st = input + lax.stop_gradient(q - input)
    return (quantize_st, loss, embed_ind)


# speedup vs baseline: 1.2447x; 1.0005x over previous
import functools

import jax
import jax.numpy as jnp
from jax import lax
from jax.experimental import pallas as pl
from jax.experimental.pallas import tpu as pltpu
from jax.experimental.pallas import tpu_sc as plsc

BETA = 0.25
N_TOK = 16384
D = 64
K = 8192
BR = 512


W = 4096


def _round_bf16(x):
    u = lax.bitcast_convert_type(x, jnp.uint32)
    r = u + jnp.uint32(0x7FFF) + ((u >> 16) & jnp.uint32(1))
    return lax.bitcast_convert_type(r & jnp.uint32(0xFFFF0000), jnp.float32)


def _argmin_body(f_ref, e_ref, a_ref, c_ref, idx_ref):
    f = f_ref[...]
    e = e_ref[...]
    mm = lax.dot_general(f, e, (((1,), (0,)), ((), ())),
                         preferred_element_type=jnp.float32)
    dist = (a_ref[...] - mm) + c_ref[...]
    d_lo = dist[:, :W]
    d_hi = dist[:, W:]
    m0 = jnp.min(d_lo, axis=1, keepdims=True)
    m1 = jnp.min(d_hi, axis=1, keepdims=True)
    take = m1 < _round_bf16(m0)
    tgt = jnp.where(take, m1, m0)
    half = jnp.where(take, d_hi, d_lo)
    ii = lax.broadcasted_iota(jnp.int32, half.shape, 1)
    wi = jnp.min(jnp.where(half == tgt, ii, K), axis=1)
    idx_ref[...] = wi + jnp.where(take[:, 0], W, 0)


_argmin = pl.pallas_call(
    _argmin_body,
    grid=(N_TOK // BR,),
    in_specs=[
        pl.BlockSpec((BR, D), lambda i: (i, 0)),
        pl.BlockSpec((D, K), lambda i: (0, 0)),
        pl.BlockSpec((BR, 1), lambda i: (i, 0)),
        pl.BlockSpec((1, K), lambda i: (0, 0)),
    ],
    out_specs=pl.BlockSpec((BR,), lambda i: (i,)),
    out_shape=jax.ShapeDtypeStruct((N_TOK,), jnp.int32),
)


def _loss_body(c_ref, f_ref, o_ref):
    d = c_ref[...] - f_ref[...]
    m = jnp.sum(d * d) / jnp.float32(N_TOK * D)
    o_ref[0, 0] = m + jnp.float32(BETA) * m


_loss = pl.pallas_call(
    _loss_body,
    in_specs=[
        pl.BlockSpec(memory_space=pltpu.VMEM),
        pl.BlockSpec(memory_space=pltpu.VMEM),
    ],
    out_specs=pl.BlockSpec(memory_space=pltpu.SMEM),
    out_shape=jax.ShapeDtypeStruct((1, 1), jnp.float32),
)


DPAD = 128


def _sc_gather(table, ind):
    mesh = plsc.VectorSubcoreMesh(core_axis_name="c", subcore_axis_name="s")
    nc, ns = mesh.num_cores, mesh.num_subcores
    nw = nc * ns
    bpw = N_TOK // nw

    @functools.partial(
        pl.kernel,
        mesh=mesh,
        out_type=jax.ShapeDtypeStruct((N_TOK, DPAD), jnp.float32),
        scratch_types=[
            pltpu.VMEM((bpw,), jnp.int32),
            pltpu.VMEM((bpw, DPAD), jnp.float32),
            pltpu.SemaphoreType.DMA,
        ],
    )
    def gather(table_hbm, idx_hbm, out_hbm, idx_v, rows_v, sem):
        wid = lax.axis_index("s") * nc + lax.axis_index("c")
        base = wid * bpw
        pltpu.sync_copy(idx_hbm.at[pl.ds(base, bpw)], idx_v)
        pltpu.async_copy(table_hbm.at[idx_v], rows_v, sem).wait()
        pltpu.sync_copy(rows_v, out_hbm.at[pl.ds(base, bpw)])

    return gather(table, ind)


def kernel(input, embeddings):
    x = jnp.transpose(input, (0, 2, 3, 1))
    flatten = x.reshape(-1, D)
    a = jnp.sum(flatten ** 2, axis=1, keepdims=True)
    c = jnp.sum(embeddings ** 2, axis=0, keepdims=True)
    f2b = (2.0 * flatten).astype(jnp.bfloat16).astype(jnp.float32)
    ind = _argmin(f2b, embeddings, a, c)
    table = jnp.pad(embeddings.T, ((0, 0), (0, DPAD - D)))
    codes = _sc_gather(table, ind)[:, :D]
    loss = _loss(codes, flatten)[0, 0]
    embed_ind = ind.reshape(x.shape[:-1])
    q = jnp.transpose(codes.reshape(x.shape), (0, 3, 1, 2))
    quantize_st = input + lax.stop_gradient(q - input)
    return (quantize_st, loss, embed_ind)
